# baseline (speedup 1.0000x reference)
import jax
import jax.numpy as jnp
from jax import lax
from jax.experimental import pallas as pl
from jax.experimental.pallas import tpu as pltpu
from jax.experimental.pallas import tpu_sc as plsc

N = 10000
E = 320000
D = 128
EPS = 1e-5

NC = 2
NS = 16
NW = NC * NS
EW = E // NW
CW = 80
CPW = EW // CW
EA = 163840
EB = E - EA
EWA = EA // NW
EWB = EB // NW
CPWA = EWA // CW
CPWB = EWB // CW
BE = 2560
NBLK = E // BE
NBLKA = EA // BE
NBLKB = EB // BE
ZCH = 200
NZCH = N // ZCH
ZB = 40


def _proj_body(h_ref, wd_ref, we_ref, dh_ref, eh_ref):
    hv = h_ref[...]
    dh_ref[...] = lax.dot(hv, wd_ref[...], preferred_element_type=jnp.float32)
    eh_ref[...] = lax.dot(hv, we_ref[...], preferred_element_type=jnp.float32)


def _proj(h, wd_t, we_t):
    return pl.pallas_call(
        _proj_body,
        out_shape=(jax.ShapeDtypeStruct((N, D), jnp.float32),
                   jax.ShapeDtypeStruct((N, D), jnp.float32)),
    )(h, wd_t, we_t)


def _make_gather(ew, cw, cpw, ne):
    def body(dh_hbm, eh_hbm, src_hbm, dst_hbm, g_hbm,
             src_v, dst_v, bufd, bufe, bufo, semd, seme, semw):
        wid = lax.axis_index("s") * NC + lax.axis_index("c")
        pltpu.sync_copy(src_hbm.at[wid], src_v)
        pltpu.sync_copy(dst_hbm.at[wid], dst_v)

        def issue(j, p):
            pltpu.async_copy(dh_hbm.at[src_v.at[j]], bufd.at[p], semd.at[p])
            pltpu.async_copy(eh_hbm.at[dst_v.at[j]], bufe.at[p], seme.at[p])

        def do_chunk(j, p):
            pltpu.make_async_copy(dh_hbm.at[src_v.at[j]], bufd.at[p],
                                  semd.at[p]).wait()
            pltpu.make_async_copy(eh_hbm.at[dst_v.at[j]], bufe.at[p],
                                  seme.at[p]).wait()

            @pl.when(j >= 2)
            def _w():
                pltpu.make_async_copy(
                    bufo.at[p], g_hbm.at[pl.ds(0, cw)], semw.at[p]).wait()

            @plsc.parallel_loop(0, cw)
            def _add(r):
                for c in range(D // 16):
                    s = pl.ds(c * 16, 16)
                    bufo[p, r, s] = bufd[p, r, s] + bufe[p, r, s]

            @pl.when(j + 2 < cpw)
            def _pf():
                issue(j + 2, p)

            pltpu.async_copy(bufo.at[p],
                             g_hbm.at[pl.ds(wid * ew + j * cw, cw)],
                             semw.at[p])

        issue(0, 0)
        issue(1, 1)

        def pair(i, carry):
            do_chunk(2 * i, 0)
            do_chunk(2 * i + 1, 1)
            return carry

        lax.fori_loop(0, cpw // 2, pair, 0)
        if cpw % 2:
            do_chunk(cpw - 1, 0)

        pltpu.make_async_copy(bufo.at[0], g_hbm.at[pl.ds(0, cw)],
                              semw.at[0]).wait()
        pltpu.make_async_copy(bufo.at[1], g_hbm.at[pl.ds(0, cw)],
                              semw.at[1]).wait()

    return pl.kernel(
        body,
        out_type=jax.ShapeDtypeStruct((ne, D), jnp.float32),
        mesh=plsc.VectorSubcoreMesh(core_axis_name="c", subcore_axis_name="s"),
        scratch_types=[
            pltpu.VMEM((cpw, cw), jnp.int32),
            pltpu.VMEM((cpw, cw), jnp.int32),
            pltpu.VMEM((2, cw, D), jnp.float32),
            pltpu.VMEM((2, cw, D), jnp.float32),
            pltpu.VMEM((2, cw, D), jnp.float32),
            pltpu.SemaphoreType.DMA((2,)),
            pltpu.SemaphoreType.DMA((2,)),
            pltpu.SemaphoreType.DMA((2,)),
        ],
    )


_gather_cache = {}


def _gather_full(dh, eh, src3d, dst3d):
    kfn = _gather_cache.get("full")
    if kfn is None:
        kfn = _make_gather(EW, CW, CPW, E)
        _gather_cache["full"] = kfn
    return kfn(dh, eh, src3d, dst3d)


def _edge1_body(cwt_ref, mwt_ref, cb_ref, mb_ref, e_ref, g_ref,
                m_ref, s_ref, ss_ref):
    i = pl.program_id(0)
    pre = lax.dot(e_ref[...], cwt_ref[...], preferred_element_type=jnp.float32)
    pre = pre + g_ref[...] + cb_ref[...]
    m = lax.dot(jnp.maximum(pre, 0.0), mwt_ref[...],
                preferred_element_type=jnp.float32) + mb_ref[...]
    m_ref[...] = m.astype(jnp.bfloat16)

    @pl.when(i == 0)
    def _init():
        s_ref[...] = jnp.zeros_like(s_ref)
        ss_ref[...] = jnp.zeros_like(ss_ref)

    s_ref[...] += jnp.sum(m, axis=0, keepdims=True)
    ss_ref[...] += jnp.sum(m * m, axis=0, keepdims=True)


def _edge1_part(e, g, cwt, mwt, cb, mb, off, nblk, ne):
    row = pl.BlockSpec((1, D), lambda i: (0, 0))
    return pl.pallas_call(
        _edge1_body,
        grid=(nblk,),
        in_specs=[
            pl.BlockSpec((D, D), lambda i: (0, 0)),
            pl.BlockSpec((D, D), lambda i: (0, 0)),
            row, row,
            pl.BlockSpec((BE, D), lambda i: (i + off, 0)),
            pl.BlockSpec((BE, D), lambda i: (i, 0)),
        ],
        out_specs=(
            pl.BlockSpec((BE, D), lambda i: (i, 0)),
            pl.BlockSpec((1, D), lambda i: (0, 0)),
            pl.BlockSpec((1, D), lambda i: (0, 0)),
        ),
        out_shape=(jax.ShapeDtypeStruct((ne, D), jnp.bfloat16),
                   jax.ShapeDtypeStruct((1, D), jnp.float32),
                   jax.ShapeDtypeStruct((1, D), jnp.float32)),
        compiler_params=pltpu.CompilerParams(
            dimension_semantics=("arbitrary",)),
    )(cwt, mwt, cb, mb, e, g)


def _edge2_body(s_ref, ss_ref, bw_ref, bb_ref, e_ref, m_ref, eo_ref):
    s = s_ref[...]
    ss = ss_ref[...]
    mean = s * (1.0 / E)
    var = ss * (1.0 / E) - mean * mean
    scale = bw_ref[...] * lax.rsqrt(var + EPS)
    shift = bb_ref[...] - mean * scale
    eo_ref[...] = e_ref[...] + m_ref[...].astype(jnp.float32) * scale + shift


def _edge2(e, m, s, ss, bw, bb):
    row = pl.BlockSpec((1, D), lambda i: (0, 0))
    return pl.pallas_call(
        _edge2_body,
        grid=(NBLK,),
        in_specs=[
            row, row, row, row,
            pl.BlockSpec((BE, D), lambda i: (i, 0)),
            pl.BlockSpec((BE, D), lambda i: (i, 0)),
        ],
        out_specs=pl.BlockSpec((BE, D), lambda i: (i, 0)),
        out_shape=jax.ShapeDtypeStruct((E, D), jnp.float32),
        compiler_params=pltpu.CompilerParams(
            dimension_semantics=("arbitrary",)),
    )(s, ss, bw, bb, e, m)


def _scatter_body(eo_hbm, dst_hbm, part_hbm, dst_v, rows_v, zbuf, h_acc, semr):
    cid = lax.axis_index("c")
    sid = lax.axis_index("s")
    wid = sid * NC + cid

    def zrow(r, carry):
        for c in range(D // 16):
            zbuf[r, pl.ds(c * 16, 16)] = jnp.zeros((16,), jnp.float32)
        return carry

    lax.fori_loop(0, ZB, zrow, 0)
    for t in range((NZCH + NS - 1) // NS):
        zc = sid + t * NS

        @pl.when(zc < NZCH)
        def _zero():
            for q in range(ZCH // ZB):
                pltpu.sync_copy(zbuf, h_acc.at[pl.ds(zc * ZCH + q * ZB, ZB)])

    plsc.subcore_barrier()

    pltpu.sync_copy(dst_hbm.at[wid], dst_v)

    def issue(j, p):
        pltpu.async_copy(eo_hbm.at[pl.ds(wid * EW + j * CW, CW)],
                         rows_v.at[p], semr.at[p])

    def do_chunk(j, p):
        pltpu.make_async_copy(eo_hbm.at[pl.ds(0, CW)], rows_v.at[p],
                              semr.at[p]).wait()
        pltpu.sync_copy(rows_v.at[p], h_acc.at[dst_v.at[j]], add=True)

        @pl.when(j + 2 < CPW)
        def _pf():
            issue(j + 2, p)

    issue(0, 0)
    issue(1, 1)

    def pair(i, carry):
        do_chunk(2 * i, 0)
        do_chunk(2 * i + 1, 1)
        return carry

    lax.fori_loop(0, CPW // 2, pair, 0)
    do_chunk(CPW - 1, 0)
    plsc.subcore_barrier()

    for t in range((NZCH + NS - 1) // NS):
        zc = sid + t * NS

        @pl.when(zc < NZCH)
        def _out():
            pltpu.sync_copy(h_acc.at[pl.ds(zc * ZCH, ZCH)],
                            part_hbm.at[pl.ds(cid * N + zc * ZCH, ZCH)])


def _scatter(e_out, dst3d):
    kfn = pl.kernel(
        _scatter_body,
        out_type=jax.ShapeDtypeStruct((NC * N, D), jnp.float32),
        mesh=plsc.VectorSubcoreMesh(core_axis_name="c", subcore_axis_name="s"),
        scratch_types=[
            pltpu.VMEM((CPW, CW), jnp.int32),
            pltpu.VMEM((2, CW, D), jnp.float32),
            pltpu.VMEM((ZB, D), jnp.float32),
            pltpu.VMEM_SHARED((N, D), jnp.float32),
            pltpu.SemaphoreType.DMA((2,)),
        ],
    )
    return kfn(e_out, dst3d)


def _bnh_body(p_ref, bw_ref, bb_ref, out_ref):
    hs = p_ref[pl.ds(0, N), :] + p_ref[pl.ds(N, N), :]
    hmean = jnp.mean(hs, axis=0, keepdims=True)
    hvar = jnp.mean(hs * hs, axis=0, keepdims=True) - hmean * hmean
    hscale = bw_ref[...] * lax.rsqrt(hvar + EPS)
    out_ref[...] = (hs - hmean) * hscale + bb_ref[...]


def _bnh(parts, bw, bb):
    return pl.pallas_call(
        _bnh_body,
        out_shape=jax.ShapeDtypeStruct((N, D), jnp.float32),
    )(parts, bw, bb)


@jax.jit
def kernel(h, e, u, edge_index, revmap, C_w, C_b, D_w, D_b, E_w, E_b,
           mlp_w, mlp_b, bn_e_w, bn_e_b, bn_h_w, bn_h_b):
    src = edge_index[0]
    dst = edge_index[1]
    src3d = src.reshape(NW, CPW, CW)
    dst3d = dst.reshape(NW, CPW, CW)

    dh, eh = _proj(h, D_w.T, E_w.T)
    g = _gather_full(dh, eh, src3d, dst3d)

    cb = (C_b + D_b + E_b).reshape(1, D)
    mb = mlp_b.reshape(1, D)
    m, s, ss = _edge1_part(e, g, C_w.T, mlp_w.T, cb, mb, 0, NBLK, E)

    e_out = _edge2(e, m, s, ss, bn_e_w.reshape(1, D), bn_e_b.reshape(1, D))

    parts = _scatter(e_out, dst3d)
    h_out = _bnh(parts, bn_h_w.reshape(1, D), bn_h_b.reshape(1, D))
    return (h_out, e_out, u)


# speedup vs baseline: 1.0378x; 1.0378x over previous
import jax
import jax.numpy as jnp
from jax import lax
from jax.experimental import pallas as pl
from jax.experimental.pallas import tpu as pltpu
from jax.experimental.pallas import tpu_sc as plsc

N = 10000
E = 320000
D = 128
EPS = 1e-5

NC = 2
NS = 16
NW = NC * NS
EW = E // NW
CW = 80
CPW = EW // CW
EA = 163840
EB = E - EA
EWA = EA // NW
EWB = EB // NW
CPWA = EWA // CW
CPWB = EWB // CW
BE = 2560
NBLK = E // BE
NBLKA = EA // BE
NBLKB = EB // BE
ZCH = 200
NZCH = N // ZCH
ZB = 40


def _proj_body(h_ref, wd_ref, we_ref, dh_ref, eh_ref):
    hv = h_ref[...]
    dh_ref[...] = lax.dot(hv, wd_ref[...], preferred_element_type=jnp.float32)
    eh_ref[...] = lax.dot(hv, we_ref[...], preferred_element_type=jnp.float32)


def _proj(h, wd_t, we_t):
    return pl.pallas_call(
        _proj_body,
        out_shape=(jax.ShapeDtypeStruct((N, D), jnp.float32),
                   jax.ShapeDtypeStruct((N, D), jnp.float32)),
    )(h, wd_t, we_t)


def _make_gather(ew, cw, cpw, ne):
    def body(dh_hbm, eh_hbm, src_hbm, dst_hbm, g_hbm,
             src_v, dst_v, bufd, bufe, bufo, semd, seme, semw):
        wid = lax.axis_index("s") * NC + lax.axis_index("c")
        pltpu.sync_copy(src_hbm.at[wid], src_v)
        pltpu.sync_copy(dst_hbm.at[wid], dst_v)

        def issue(j, p):
            pltpu.async_copy(dh_hbm.at[src_v.at[j]], bufd.at[p], semd.at[p])
            pltpu.async_copy(eh_hbm.at[dst_v.at[j]], bufe.at[p], seme.at[p])

        def do_chunk(j, p):
            pltpu.make_async_copy(dh_hbm.at[src_v.at[j]], bufd.at[p],
                                  semd.at[p]).wait()
            pltpu.make_async_copy(eh_hbm.at[dst_v.at[j]], bufe.at[p],
                                  seme.at[p]).wait()

            @pl.when(j >= 2)
            def _w():
                pltpu.make_async_copy(
                    bufo.at[p], g_hbm.at[pl.ds(0, cw)], semw.at[p]).wait()

            @plsc.parallel_loop(0, cw)
            def _add(r):
                for c in range(D // 16):
                    s = pl.ds(c * 16, 16)
                    bufo[p, r, s] = bufd[p, r, s] + bufe[p, r, s]

            @pl.when(j + 2 < cpw)
            def _pf():
                issue(j + 2, p)

            pltpu.async_copy(bufo.at[p],
                             g_hbm.at[pl.ds(wid * ew + j * cw, cw)],
                             semw.at[p])

        issue(0, 0)
        issue(1, 1)

        def pair(i, carry):
            do_chunk(2 * i, 0)
            do_chunk(2 * i + 1, 1)
            return carry

        lax.fori_loop(0, cpw // 2, pair, 0)
        if cpw % 2:
            do_chunk(cpw - 1, 0)

        pltpu.make_async_copy(bufo.at[0], g_hbm.at[pl.ds(0, cw)],
                              semw.at[0]).wait()
        pltpu.make_async_copy(bufo.at[1], g_hbm.at[pl.ds(0, cw)],
                              semw.at[1]).wait()

    return pl.kernel(
        body,
        out_type=jax.ShapeDtypeStruct((ne, D), jnp.float32),
        mesh=plsc.VectorSubcoreMesh(core_axis_name="c", subcore_axis_name="s"),
        scratch_types=[
            pltpu.VMEM((cpw, cw), jnp.int32),
            pltpu.VMEM((cpw, cw), jnp.int32),
            pltpu.VMEM((2, cw, D), jnp.float32),
            pltpu.VMEM((2, cw, D), jnp.float32),
            pltpu.VMEM((2, cw, D), jnp.float32),
            pltpu.SemaphoreType.DMA((2,)),
            pltpu.SemaphoreType.DMA((2,)),
            pltpu.SemaphoreType.DMA((2,)),
        ],
    )


_gather_cache = {}


def _gather_part(dh, eh, src3d, dst3d, which):
    kfn = _gather_cache.get(which)
    if kfn is None:
        if which == "A":
            kfn = _make_gather(EWA, CW, CPWA, EA)
        else:
            kfn = _make_gather(EWB, CW, CPWB, EB)
        _gather_cache[which] = kfn
    return kfn(dh, eh, src3d, dst3d)


def _edge1_body(cwt_ref, mwt_ref, cb_ref, mb_ref, e_ref, g_ref,
                m_ref, s_ref, ss_ref):
    i = pl.program_id(0)
    pre = lax.dot(e_ref[...], cwt_ref[...], preferred_element_type=jnp.float32)
    pre = pre + g_ref[...] + cb_ref[...]
    m = lax.dot(jnp.maximum(pre, 0.0), mwt_ref[...],
                preferred_element_type=jnp.float32) + mb_ref[...]
    m_ref[...] = m.astype(jnp.bfloat16)

    @pl.when(i == 0)
    def _init():
        s_ref[...] = jnp.zeros_like(s_ref)
        ss_ref[...] = jnp.zeros_like(ss_ref)

    s_ref[...] += jnp.sum(m, axis=0, keepdims=True)
    ss_ref[...] += jnp.sum(m * m, axis=0, keepdims=True)


def _edge1_part(e, g, cwt, mwt, cb, mb, off, nblk, ne):
    row = pl.BlockSpec((1, D), lambda i: (0, 0))
    return pl.pallas_call(
        _edge1_body,
        grid=(nblk,),
        in_specs=[
            pl.BlockSpec((D, D), lambda i: (0, 0)),
            pl.BlockSpec((D, D), lambda i: (0, 0)),
            row, row,
            pl.BlockSpec((BE, D), lambda i: (i + off, 0)),
            pl.BlockSpec((BE, D), lambda i: (i, 0)),
        ],
        out_specs=(
            pl.BlockSpec((BE, D), lambda i: (i, 0)),
            pl.BlockSpec((1, D), lambda i: (0, 0)),
            pl.BlockSpec((1, D), lambda i: (0, 0)),
        ),
        out_shape=(jax.ShapeDtypeStruct((ne, D), jnp.bfloat16),
                   jax.ShapeDtypeStruct((1, D), jnp.float32),
                   jax.ShapeDtypeStruct((1, D), jnp.float32)),
        compiler_params=pltpu.CompilerParams(
            dimension_semantics=("arbitrary",)),
    )(cwt, mwt, cb, mb, e, g)


def _edge2_body(sa_ref, ssa_ref, sb_ref, ssb_ref, bw_ref, bb_ref,
                e_ref, ma_ref, mb_ref, eo_ref):
    i = pl.program_id(0)
    s = sa_ref[...] + sb_ref[...]
    ss = ssa_ref[...] + ssb_ref[...]
    mean = s * (1.0 / E)
    var = ss * (1.0 / E) - mean * mean
    scale = bw_ref[...] * lax.rsqrt(var + EPS)
    shift = bb_ref[...] - mean * scale
    mv = jnp.where(i < NBLKA, ma_ref[...], mb_ref[...]).astype(jnp.float32)
    eo_ref[...] = e_ref[...] + mv * scale + shift


def _edge2(e, ma, mb, sa, ssa, sb, ssb, bw, bb):
    row = pl.BlockSpec((1, D), lambda i: (0, 0))
    return pl.pallas_call(
        _edge2_body,
        grid=(NBLK,),
        in_specs=[
            row, row, row, row, row, row,
            pl.BlockSpec((BE, D), lambda i: (i, 0)),
            pl.BlockSpec((BE, D),
                         lambda i: (jnp.minimum(i, NBLKA - 1), 0)),
            pl.BlockSpec((BE, D),
                         lambda i: (jnp.maximum(i - NBLKA, 0), 0)),
        ],
        out_specs=pl.BlockSpec((BE, D), lambda i: (i, 0)),
        out_shape=jax.ShapeDtypeStruct((E, D), jnp.float32),
        compiler_params=pltpu.CompilerParams(
            dimension_semantics=("arbitrary",)),
    )(sa, ssa, sb, ssb, bw, bb, e, ma, mb)


def _scatter_body(eo_hbm, dst_hbm, part_hbm, dst_v, rows_v, zbuf, h_acc, semr):
    cid = lax.axis_index("c")
    sid = lax.axis_index("s")
    wid = sid * NC + cid

    def zrow(r, carry):
        for c in range(D // 16):
            zbuf[r, pl.ds(c * 16, 16)] = jnp.zeros((16,), jnp.float32)
        return carry

    lax.fori_loop(0, ZB, zrow, 0)
    for t in range((NZCH + NS - 1) // NS):
        zc = sid + t * NS

        @pl.when(zc < NZCH)
        def _zero():
            for q in range(ZCH // ZB):
                pltpu.sync_copy(zbuf, h_acc.at[pl.ds(zc * ZCH + q * ZB, ZB)])

    plsc.subcore_barrier()

    pltpu.sync_copy(dst_hbm.at[wid], dst_v)

    def issue(j, p):
        pltpu.async_copy(eo_hbm.at[pl.ds(wid * EW + j * CW, CW)],
                         rows_v.at[p], semr.at[p])

    def do_chunk(j, p):
        pltpu.make_async_copy(eo_hbm.at[pl.ds(0, CW)], rows_v.at[p],
                              semr.at[p]).wait()
        pltpu.sync_copy(rows_v.at[p], h_acc.at[dst_v.at[j]], add=True)

        @pl.when(j + 2 < CPW)
        def _pf():
            issue(j + 2, p)

    issue(0, 0)
    issue(1, 1)

    def pair(i, carry):
        do_chunk(2 * i, 0)
        do_chunk(2 * i + 1, 1)
        return carry

    lax.fori_loop(0, CPW // 2, pair, 0)
    do_chunk(CPW - 1, 0)
    plsc.subcore_barrier()

    for t in range((NZCH + NS - 1) // NS):
        zc = sid + t * NS

        @pl.when(zc < NZCH)
        def _out():
            pltpu.sync_copy(h_acc.at[pl.ds(zc * ZCH, ZCH)],
                            part_hbm.at[pl.ds(cid * N + zc * ZCH, ZCH)])


def _scatter(e_out, dst3d):
    kfn = pl.kernel(
        _scatter_body,
        out_type=jax.ShapeDtypeStruct((NC * N, D), jnp.float32),
        mesh=plsc.VectorSubcoreMesh(core_axis_name="c", subcore_axis_name="s"),
        scratch_types=[
            pltpu.VMEM((CPW, CW), jnp.int32),
            pltpu.VMEM((2, CW, D), jnp.float32),
            pltpu.VMEM((ZB, D), jnp.float32),
            pltpu.VMEM_SHARED((N, D), jnp.float32),
            pltpu.SemaphoreType.DMA((2,)),
        ],
    )
    return kfn(e_out, dst3d)


def _bnh_body(p_ref, bw_ref, bb_ref, out_ref):
    hs = p_ref[pl.ds(0, N), :] + p_ref[pl.ds(N, N), :]
    hmean = jnp.mean(hs, axis=0, keepdims=True)
    hvar = jnp.mean(hs * hs, axis=0, keepdims=True) - hmean * hmean
    hscale = bw_ref[...] * lax.rsqrt(hvar + EPS)
    out_ref[...] = (hs - hmean) * hscale + bb_ref[...]


def _bnh(parts, bw, bb):
    return pl.pallas_call(
        _bnh_body,
        out_shape=jax.ShapeDtypeStruct((N, D), jnp.float32),
    )(parts, bw, bb)


@jax.jit
def kernel(h, e, u, edge_index, revmap, C_w, C_b, D_w, D_b, E_w, E_b,
           mlp_w, mlp_b, bn_e_w, bn_e_b, bn_h_w, bn_h_b):
    src = edge_index[0]
    dst = edge_index[1]
    srcA = src[:EA].reshape(NW, CPWA, CW)
    dstA = dst[:EA].reshape(NW, CPWA, CW)
    srcB = src[EA:].reshape(NW, CPWB, CW)
    dstB = dst[EA:].reshape(NW, CPWB, CW)
    dst3d = dst.reshape(NW, CPW, CW)

    dh, eh = _proj(h, D_w.T, E_w.T)
    gA = _gather_part(dh, eh, srcA, dstA, "A")
    gB = _gather_part(dh, eh, srcB, dstB, "B")

    cb = (C_b + D_b + E_b).reshape(1, D)
    mb = mlp_b.reshape(1, D)
    mA, sA, ssA = _edge1_part(e, gA, C_w.T, mlp_w.T, cb, mb, 0, NBLKA, EA)
    mB, sB, ssB = _edge1_part(e, gB, C_w.T, mlp_w.T, cb, mb, NBLKA, NBLKB, EB)

    e_out = _edge2(e, mA, mB, sA, ssA, sB, ssB,
                   bn_e_w.reshape(1, D), bn_e_b.reshape(1, D))

    parts = _scatter(e_out, dst3d)
    h_out = _bnh(parts, bn_h_w.reshape(1, D), bn_h_b.reshape(1, D))
    return (h_out, e_out, u)
